# baseline (speedup 1.0000x reference)
import functools

import jax
import jax.numpy as jnp
from jax import lax
from jax.experimental import pallas as pl
from jax.experimental.pallas import tpu as pltpu
from jax.experimental.pallas import tpu_sc as plsc

NC = 2
NS = 16
NW = NC * NS

RPC = 4
NBUF = 2


def _emb_body(ids_hbm, table_hbm, out_hbm, idx_v, rows_v, sem_g, sem_o):
    n_batch, seq = ids_hbm.shape
    d = table_hbm.shape[1]
    bpw = n_batch // NW
    n_chunks = bpw // RPC
    n_grp = n_chunks // NBUF
    wid = lax.axis_index("s") * NC + lax.axis_index("c")
    brow0 = wid * bpw
    s0 = (seq // 2 // 8) * 8
    s0 = 128
    s1 = seq - s0

    def start_gather(j, b):
        brow = brow0 + j * RPC
        pltpu.sync_copy(ids_hbm.at[pl.ds(brow, RPC)], idx_v.at[b])
        for r in range(RPC):
            pltpu.async_copy(
                table_hbm.at[idx_v.at[b].at[r].at[pl.ds(0, s0)]],
                rows_v.at[b].at[r].at[pl.ds(0, s0)],
                sem_g[b],
            )
            pltpu.async_copy(
                table_hbm.at[idx_v.at[b].at[r].at[pl.ds(s0, s1)]],
                rows_v.at[b].at[r].at[pl.ds(s0, s1)],
                sem_g[b],
            )

    def wait_gather(b):
        for r in range(RPC):
            pltpu.make_async_copy(
                table_hbm.at[pl.ds(0, s0)],
                rows_v.at[b].at[r].at[pl.ds(0, s0)],
                sem_g[b],
            ).wait()
            pltpu.make_async_copy(
                table_hbm.at[pl.ds(0, s1)],
                rows_v.at[b].at[r].at[pl.ds(s0, s1)],
                sem_g[b],
            ).wait()

    def start_wb(j, b):
        brow = brow0 + j * RPC
        pltpu.async_copy(rows_v.at[b], out_hbm.at[pl.ds(brow, RPC)], sem_o[b])

    def wait_wb(b):
        pltpu.make_async_copy(
            rows_v.at[b], out_hbm.at[pl.ds(0, RPC)], sem_o[b]
        ).wait()

    for b in range(NBUF):
        start_gather(b, b)

    def grp(g, _):
        for b in range(NBUF):
            j = g * NBUF + b
            wait_gather(b)
            start_wb(j, b)
            wait_wb(b)
            start_gather(j + NBUF, b)
        return 0

    lax.fori_loop(0, n_grp - 1, grp, 0)

    for b in range(NBUF):
        j = (n_grp - 1) * NBUF + b
        wait_gather(b)
        start_wb(j, b)
        wait_wb(b)


@functools.partial(jax.jit, static_argnames=())
def kernel(input_ids, table):
    batch, seq_len = input_ids.shape
    d = table.shape[1]

    mesh = plsc.VectorSubcoreMesh(core_axis_name="c", subcore_axis_name="s")
    out = pl.kernel(
        _emb_body,
        out_type=jax.ShapeDtypeStruct((batch, seq_len, d), jnp.float32),
        mesh=mesh,
        scratch_types=[
            pltpu.VMEM((NBUF, RPC, seq_len), jnp.int32),
            pltpu.VMEM((NBUF, RPC, seq_len, d), jnp.float32),
            [pltpu.SemaphoreType.DMA] * NBUF,
            [pltpu.SemaphoreType.DMA] * NBUF,
        ],
        compiler_params=pltpu.CompilerParams(use_tc_tiling_on_sc=False),
    )(input_ids, table)
    return out


# speedup vs baseline: 1.2238x; 1.2238x over previous
import functools

import jax
import jax.numpy as jnp
from jax import lax
from jax.experimental import pallas as pl
from jax.experimental.pallas import tpu as pltpu
from jax.experimental.pallas import tpu_sc as plsc

NC = 2
NS = 16
NW = NC * NS

IDXW = 128
GPC = 2
CHUNK = IDXW * GPC
NBUF = 2


def _emb_body(ids_hbm, table_hbm, out_hbm, idx_v, rows_v, sem_g, sem_o):
    dp = table_hbm.shape[1]
    b_total = out_hbm.shape[0]
    b_per_w = b_total // NW
    n_chunks = b_per_w // CHUNK
    n_grp = n_chunks // NBUF
    wid = lax.axis_index("s") * NC + lax.axis_index("c")
    row0 = wid * (b_per_w // IDXW)
    base0 = wid * b_per_w

    def start_gather(j, b):
        pltpu.sync_copy(ids_hbm.at[pl.ds(row0 + j * GPC, GPC)], idx_v.at[b])
        for r in range(GPC):
            pltpu.async_copy(
                table_hbm.at[idx_v.at[b].at[r]],
                rows_v.at[b].at[pl.ds(r * IDXW, IDXW)],
                sem_g[b],
            )

    def wait_gather(b):
        for r in range(GPC):
            pltpu.make_async_copy(
                table_hbm.at[pl.ds(0, IDXW)],
                rows_v.at[b].at[pl.ds(r * IDXW, IDXW)],
                sem_g[b],
            ).wait()

    def start_wb(j, b):
        pltpu.async_copy(
            rows_v.at[b], out_hbm.at[pl.ds(base0 + j * CHUNK, CHUNK)], sem_o[b]
        )

    def wait_wb(b):
        pltpu.make_async_copy(
            rows_v.at[b], out_hbm.at[pl.ds(0, CHUNK)], sem_o[b]
        ).wait()

    for b in range(NBUF):
        start_gather(b, b)

    def grp(g, _):
        for b in range(NBUF):
            j = g * NBUF + b
            wait_gather(b)
            start_wb(j, b)
            wait_wb(b)
            start_gather(j + NBUF, b)
        return 0

    lax.fori_loop(0, n_grp - 1, grp, 0)

    for b in range(NBUF):
        j = (n_grp - 1) * NBUF + b
        wait_gather(b)
        start_wb(j, b)
        wait_wb(b)


@functools.partial(jax.jit, static_argnames=())
def kernel(input_ids, table):
    batch, seq_len = input_ids.shape
    v, d = table.shape
    b = batch * seq_len
    dp = 128

    ids2d = input_ids.reshape(b // IDXW, IDXW)
    tpad = jnp.pad(table, ((0, 0), (0, dp - d)))

    mesh = plsc.VectorSubcoreMesh(core_axis_name="c", subcore_axis_name="s")
    out = pl.kernel(
        _emb_body,
        out_type=jax.ShapeDtypeStruct((b, dp), jnp.float32),
        mesh=mesh,
        scratch_types=[
            pltpu.VMEM((NBUF, GPC, IDXW), jnp.int32),
            pltpu.VMEM((NBUF, CHUNK, dp), jnp.float32),
            [pltpu.SemaphoreType.DMA] * NBUF,
            [pltpu.SemaphoreType.DMA] * NBUF,
        ],
        compiler_params=pltpu.CompilerParams(use_tc_tiling_on_sc=False),
    )(ids2d, tpad)
    return out[:, :d].reshape(batch, seq_len, d)
